# baseline (speedup 1.0000x reference)
import math

import jax
import jax.numpy as jnp
from jax import lax
from jax.experimental import pallas as pl
from jax.experimental.pallas import tpu as pltpu

_B, _N, _D = 8, 512, 128
_IOU_THRESHOLD = 0.8
_INV_TEMP = 10.0
_LOG2E = math.log2(math.e)
_SIDE_SCALE = math.sqrt(_INV_TEMP * _LOG2E)


def _one_batch(b, iou_b, sim_b, an_ref, bn_ref):
    rowmax = jnp.max(iou_b, axis=-1, keepdims=True)
    col = lax.broadcasted_iota(jnp.int32, (_N, _N), 1)
    eq = iou_b == rowmax
    idx = jnp.min(jnp.where(eq, col, _N), axis=-1, keepdims=True)
    onehot = (col == idx).astype(jnp.float32)
    pos = jnp.sum(onehot * sim_b, axis=-1)

    an_b = an_ref[pl.ds(b * _N, _N), :]
    match = jnp.dot(onehot.astype(jnp.bfloat16), an_b.astype(jnp.bfloat16),
                    preferred_element_type=jnp.float32)
    m8 = match.astype(jnp.float8_e4m3fn)

    acc = jnp.zeros((_N, _D), jnp.float32)
    for j in range(_B - 1):
        jj = j + (j >= b).astype(jnp.int32)
        a_j = an_ref[pl.ds(jj * _N, _N), :]
        b_j = bn_ref[pl.ds(jj * _N, _N), :]
        ga = lax.dot_general(m8, a_j, (((1,), (1,)), ((), ())),
                             preferred_element_type=jnp.float32)
        gb = lax.dot_general(m8, b_j, (((1,), (1,)), ((), ())),
                             preferred_element_type=jnp.float32)
        ea = jnp.exp2(ga.astype(jnp.bfloat16))
        eb = jnp.exp2(gb.astype(jnp.bfloat16))
        sa = ((ea[:, 0:128] + ea[:, 128:256])
              + (ea[:, 256:384] + ea[:, 384:512]))
        sb = ((eb[:, 0:128] + eb[:, 128:256])
              + (eb[:, 256:384] + eb[:, 384:512]))
        acc = acc + (sa.astype(jnp.float32) + sb.astype(jnp.float32))
    total = jnp.sum(acc, axis=-1) + jnp.exp2(pos * (_INV_TEMP * _LOG2E))

    row_loss = jnp.log(total) - pos * _INV_TEMP
    rm = (rowmax[:, 0] >= _IOU_THRESHOLD).astype(jnp.float32)
    cnt = jnp.sum(rm)
    return jnp.sum(row_loss * rm) / cnt, cnt.astype(jnp.int32)


def _norm_kernel(feat_a_ref, feat_b_ref, an_ref, bn_ref):
    fa = feat_a_ref[...].reshape(_B * _N, _D)
    fb = feat_b_ref[...].reshape(_B * _N, _D)
    na = jnp.sqrt(jnp.sum(fa * fa, axis=-1, keepdims=True)) + 1e-8
    nb = jnp.sqrt(jnp.sum(fb * fb, axis=-1, keepdims=True)) + 1e-8
    an_ref[...] = (fa * (_SIDE_SCALE / na)).astype(jnp.float8_e4m3fn)
    bn_ref[...] = (fb * (_SIDE_SCALE / nb)).astype(jnp.float8_e4m3fn)


def _loss_kernel(an_ref, bn_ref, sim_ref, iou_ref, loss_ref, cnt_ref):
    g = pl.program_id(0)
    l0, c0 = _one_batch(2 * g, iou_ref[0], sim_ref[0], an_ref, bn_ref)
    l1, c1 = _one_batch(2 * g + 1, iou_ref[1], sim_ref[1], an_ref, bn_ref)
    loss_ref[...] = jnp.stack([l0, l1])[:, None, None]
    cnt_ref[...] = jnp.stack([c0, c1])[:, None, None]


def kernel(feat_a, feat_b, sim, iou):
    an, bn = pl.pallas_call(
        _norm_kernel,
        out_shape=[
            jax.ShapeDtypeStruct((_B * _N, _D), jnp.float8_e4m3fn),
            jax.ShapeDtypeStruct((_B * _N, _D), jnp.float8_e4m3fn),
        ],
    )(feat_a, feat_b)
    loss, cnt = pl.pallas_call(
        _loss_kernel,
        grid=(_B // 2,),
        compiler_params=pltpu.CompilerParams(
            dimension_semantics=("parallel",)),
        in_specs=[
            pl.BlockSpec((_B * _N, _D), lambda g: (0, 0)),
            pl.BlockSpec((_B * _N, _D), lambda g: (0, 0)),
            pl.BlockSpec((2, _N, _N), lambda g: (g, 0, 0)),
            pl.BlockSpec((2, _N, _N), lambda g: (g, 0, 0)),
        ],
        out_specs=[
            pl.BlockSpec((2, 1, 1), lambda g: (g, 0, 0)),
            pl.BlockSpec((2, 1, 1), lambda g: (g, 0, 0)),
        ],
        out_shape=[
            jax.ShapeDtypeStruct((_B, 1, 1), jnp.float32),
            jax.ShapeDtypeStruct((_B, 1, 1), jnp.int32),
        ],
    )(an, bn, sim, iou)
    return (loss[:, 0, 0], cnt[:, 0, 0])


# speedup vs baseline: 1.0764x; 1.0764x over previous
import math

import jax
import jax.numpy as jnp
from jax import lax
from jax.experimental import pallas as pl
from jax.experimental.pallas import tpu as pltpu

_B, _N, _D = 8, 512, 128
_IOU_THRESHOLD = 0.8
_INV_TEMP = 10.0
_LOG2E = math.log2(math.e)
_SIDE_SCALE = math.sqrt(_INV_TEMP * _LOG2E)


def _one_batch(b, iou_b, sim_b, an_ref, bn_ref):
    rowmax = jnp.max(iou_b, axis=-1, keepdims=True)
    col = lax.broadcasted_iota(jnp.int32, (_N, _N), 1)
    eq = iou_b == rowmax
    idx = jnp.min(jnp.where(eq, col, _N), axis=-1, keepdims=True)
    onehot = (col == idx).astype(jnp.float32)
    pos = jnp.sum(onehot * sim_b, axis=-1)

    an_b = an_ref[pl.ds(b * _N, _N), :]
    match = jnp.dot(onehot.astype(jnp.bfloat16), an_b.astype(jnp.bfloat16),
                    preferred_element_type=jnp.float32)
    m8 = match.astype(jnp.float8_e4m3fn)

    acc = jnp.zeros((_N, _D), jnp.float32)
    for j in range(_B - 1):
        jj = j + (j >= b).astype(jnp.int32)
        a_j = an_ref[pl.ds(jj * _N, _N), :]
        b_j = bn_ref[pl.ds(jj * _N, _N), :]
        ga = lax.dot_general(m8, a_j, (((1,), (1,)), ((), ())),
                             preferred_element_type=jnp.float32)
        gb = lax.dot_general(m8, b_j, (((1,), (1,)), ((), ())),
                             preferred_element_type=jnp.float32)
        ea = jnp.exp2(ga.astype(jnp.bfloat16))
        eb = jnp.exp2(gb.astype(jnp.bfloat16))
        sa = ((ea[:, 0:128] + ea[:, 128:256])
              + (ea[:, 256:384] + ea[:, 384:512]))
        sb = ((eb[:, 0:128] + eb[:, 128:256])
              + (eb[:, 256:384] + eb[:, 384:512]))
        acc = acc + (sa.astype(jnp.float32) + sb.astype(jnp.float32))
    total = jnp.sum(acc, axis=-1) + jnp.exp2(pos * (_INV_TEMP * _LOG2E))

    row_loss = jnp.log(total) - pos * _INV_TEMP
    rm = (rowmax[:, 0] >= _IOU_THRESHOLD).astype(jnp.float32)
    cnt = jnp.sum(rm)
    return jnp.sum(row_loss * rm) / cnt, cnt.astype(jnp.int32)


def _loss_kernel(feat_a_ref, feat_b_ref, sim_ref, iou_ref,
                 loss_ref, cnt_ref, an_ref, bn_ref):
    g = pl.program_id(0)

    @pl.when(g == 0)
    def _():
        fa = feat_a_ref[...].reshape(_B * _N, _D)
        fb = feat_b_ref[...].reshape(_B * _N, _D)
        na = jnp.sqrt(jnp.sum(fa * fa, axis=-1, keepdims=True)) + 1e-8
        nb = jnp.sqrt(jnp.sum(fb * fb, axis=-1, keepdims=True)) + 1e-8
        an_ref[...] = (fa * (_SIDE_SCALE / na)).astype(jnp.float8_e4m3fn)
        bn_ref[...] = (fb * (_SIDE_SCALE / nb)).astype(jnp.float8_e4m3fn)

    l0, c0 = _one_batch(g, iou_ref[0], sim_ref[0], an_ref, bn_ref)
    loss_ref[...] = l0[None, None, None]
    cnt_ref[...] = c0[None, None, None]


def kernel(feat_a, feat_b, sim, iou):
    loss, cnt = pl.pallas_call(
        _loss_kernel,
        grid=(_B,),
        in_specs=[
            pl.BlockSpec((_B, _N, _D), lambda g: (0, 0, 0)),
            pl.BlockSpec((_B, _N, _D), lambda g: (0, 0, 0)),
            pl.BlockSpec((1, _N, _N), lambda g: (g, 0, 0)),
            pl.BlockSpec((1, _N, _N), lambda g: (g, 0, 0)),
        ],
        out_specs=[
            pl.BlockSpec((1, 1, 1), lambda g: (g, 0, 0)),
            pl.BlockSpec((1, 1, 1), lambda g: (g, 0, 0)),
        ],
        out_shape=[
            jax.ShapeDtypeStruct((_B, 1, 1), jnp.float32),
            jax.ShapeDtypeStruct((_B, 1, 1), jnp.int32),
        ],
        scratch_shapes=[
            pltpu.VMEM((_B * _N, _D), jnp.float8_e4m3fn),
            pltpu.VMEM((_B * _N, _D), jnp.float8_e4m3fn),
        ],
    )(feat_a, feat_b, sim, iou)
    return (loss[:, 0, 0], cnt[:, 0, 0])
